# Initial kernel scaffold; baseline (speedup 1.0000x reference)
import jax
import jax.numpy as jnp
from jax.experimental import pallas as pl


def kernel(f_atoms, f_bonds, a2b, b2a, b2revb, a_scope, W_i, W_h, W_o, b_o):
    raise NotImplementedError("write your pallas kernel here")


# speedup vs baseline: 1.0210x; 1.0210x over previous
import functools

import jax
import jax.numpy as jnp
from jax import lax
from jax.experimental import pallas as pl
from jax.experimental.pallas import tpu as pltpu
from jax.experimental.pallas import tpu_sc as plsc

A = 10000
E = 320000
NB = 32
H = 300
HP = 384
FA = 133
FB = 14
M = 500
APM = 20
DEPTH = 3

NC = 2
NS = 16
NW = NC * NS

A2 = 10240
PW_E = E // NW
PW_A = A2 // NW
C1 = 80
CA = 4

_mesh = plsc.VectorSubcoreMesh(core_axis_name="c", subcore_axis_name="s")


def _wid():
    return lax.axis_index("s") * NC + lax.axis_index("c")


@functools.partial(
    pl.kernel, mesh=_mesh,
    out_type=jax.ShapeDtypeStruct((E, HP), jnp.float32),
    scratch_types=[
        pltpu.VMEM((C1,), jnp.int32),
        pltpu.VMEM((C1, HP), jnp.float32),
        pltpu.SemaphoreType.DMA,
    ],
)
def _sc_gather_rows(table, idx, out, idx_v, buf, sem):
    wid = _wid()

    def body(i, carry):
        base = pl.multiple_of(wid * PW_E + i * C1, C1)
        pltpu.sync_copy(idx.at[pl.ds(base, C1)], idx_v)
        pltpu.async_copy(table.at[idx_v], buf, sem).wait()
        pltpu.sync_copy(buf, out.at[pl.ds(base, C1)])
        return carry

    lax.fori_loop(0, PW_E // C1, body, 0)


@functools.partial(
    pl.kernel, mesh=_mesh,
    out_type=jax.ShapeDtypeStruct((A2, HP), jnp.float32),
    scratch_types=[
        pltpu.VMEM((CA * NB,), jnp.int32),
        pltpu.VMEM((CA * NB, HP), jnp.float32),
        pltpu.VMEM((CA, HP), jnp.float32),
        pltpu.SemaphoreType.DMA,
    ],
)
def _sc_gather_sum(msg, a2b_flat, out, idx_v, gbuf, obuf, sem):
    wid = _wid()

    def body(i, carry):
        abase = pl.multiple_of(wid * PW_A + i * CA, CA)
        pltpu.sync_copy(a2b_flat.at[pl.ds(abase * NB, CA * NB)], idx_v)
        pltpu.async_copy(msg.at[idx_v], gbuf, sem).wait()

        def tbody(t, c):
            off = t * 16
            for a in range(CA):
                acc = gbuf[a * NB, pl.ds(off, 16)]
                for k in range(1, NB):
                    acc = acc + gbuf[a * NB + k, pl.ds(off, 16)]
                obuf[a, pl.ds(off, 16)] = acc
            return c

        lax.fori_loop(0, HP // 16, tbody, 0)
        pltpu.sync_copy(obuf, out.at[pl.ds(abase, CA)])
        return carry

    lax.fori_loop(0, PW_A // CA, body, 0)


@functools.partial(
    pl.kernel, mesh=_mesh,
    out_type=jax.ShapeDtypeStruct((E, HP), jnp.float32),
    scratch_types=[
        pltpu.VMEM((C1,), jnp.int32),
        pltpu.VMEM((C1,), jnp.int32),
        pltpu.VMEM((C1, HP), jnp.float32),
        pltpu.VMEM((C1, HP), jnp.float32),
        pltpu.SemaphoreType.DMA,
        pltpu.SemaphoreType.DMA,
    ],
)
def _sc_gather_sub(neia, msg, b2a, b2revb, out, idx1, idx2, bufa, bufb,
                   sema, semb):
    wid = _wid()

    def body(i, carry):
        base = pl.multiple_of(wid * PW_E + i * C1, C1)
        pltpu.sync_copy(b2a.at[pl.ds(base, C1)], idx1)
        pltpu.sync_copy(b2revb.at[pl.ds(base, C1)], idx2)
        cpa = pltpu.async_copy(neia.at[idx1], bufa, sema)
        cpb = pltpu.async_copy(msg.at[idx2], bufb, semb)
        cpa.wait()
        cpb.wait()

        def rbody(r, c):
            for t in range(HP // 16):
                sl = pl.ds(t * 16, 16)
                bufa[r, sl] = bufa[r, sl] - bufb[r, sl]
            return c

        lax.fori_loop(0, C1, rbody, 0)
        pltpu.sync_copy(bufa, out.at[pl.ds(base, C1)])
        return carry

    lax.fori_loop(0, PW_E // C1, body, 0)


def _mm_body(x_ref, w_ref, o_ref):
    o_ref[...] = jnp.dot(x_ref[...], w_ref[...],
                         preferred_element_type=jnp.float32)


def _add_mm_relu_body(g_ref, x_ref, w_ref, o_ref):
    acc = g_ref[...] + jnp.dot(x_ref[...], w_ref[...],
                               preferred_element_type=jnp.float32)
    o_ref[...] = jnp.maximum(acc, 0.0)


def _mm_relu_body(x_ref, w_ref, o_ref):
    acc = jnp.dot(x_ref[...], w_ref[...], preferred_element_type=jnp.float32)
    o_ref[...] = jnp.maximum(acc, 0.0)


def _final_body(fa_ref, am_ref, w1_ref, w2_ref, b_ref, o_ref):
    hid = jnp.dot(fa_ref[...], w1_ref[...],
                  preferred_element_type=jnp.float32)
    hid = hid + jnp.dot(am_ref[...], w2_ref[...],
                        preferred_element_type=jnp.float32)
    hid = jnp.maximum(hid + b_ref[...], 0.0)
    BM = hid.shape[0] // APM
    mean = jnp.mean(hid.reshape(BM, APM, HP), axis=1)
    o_ref[...] = mean[:, :H]


_BR = 512

_t_atom_mm = pl.pallas_call(
    _mm_body,
    grid=(A2 // _BR,),
    in_specs=[pl.BlockSpec((_BR, FA), lambda i: (i, 0)),
              pl.BlockSpec((FA, HP), lambda i: (0, 0))],
    out_specs=pl.BlockSpec((_BR, HP), lambda i: (i, 0)),
    out_shape=jax.ShapeDtypeStruct((A2, HP), jnp.float32),
)

_t_init = pl.pallas_call(
    _add_mm_relu_body,
    grid=(E // _BR,),
    in_specs=[pl.BlockSpec((_BR, HP), lambda i: (i, 0)),
              pl.BlockSpec((_BR, FB), lambda i: (i, 0)),
              pl.BlockSpec((FB, HP), lambda i: (0, 0))],
    out_specs=pl.BlockSpec((_BR, HP), lambda i: (i, 0)),
    out_shape=jax.ShapeDtypeStruct((E, HP), jnp.float32),
)

_t_update = pl.pallas_call(
    _mm_relu_body,
    grid=(E // _BR,),
    in_specs=[pl.BlockSpec((_BR, HP), lambda i: (i, 0)),
              pl.BlockSpec((HP, HP), lambda i: (0, 0))],
    out_specs=pl.BlockSpec((_BR, HP), lambda i: (i, 0)),
    out_shape=jax.ShapeDtypeStruct((E, HP), jnp.float32),
)

_t_final = pl.pallas_call(
    _final_body,
    grid=(1,),
    in_specs=[pl.BlockSpec((A, FA), lambda i: (0, 0)),
              pl.BlockSpec((A, HP), lambda i: (0, 0)),
              pl.BlockSpec((FA, HP), lambda i: (0, 0)),
              pl.BlockSpec((HP, HP), lambda i: (0, 0)),
              pl.BlockSpec((1, HP), lambda i: (0, 0))],
    out_specs=pl.BlockSpec((M, H), lambda i: (0, 0)),
    out_shape=jax.ShapeDtypeStruct((M, H), jnp.float32),
)


def kernel(f_atoms, f_bonds, a2b, b2a, b2revb, a_scope, W_i, W_h, W_o, b_o):
    f32 = jnp.float32
    wi1 = jnp.pad(W_i[:FA], ((0, 0), (0, HP - H))).astype(f32)
    wi2 = jnp.pad(W_i[FA:], ((0, 0), (0, HP - H))).astype(f32)
    whp = jnp.pad(W_h, ((0, HP - H), (0, HP - H))).astype(f32)
    wo1 = jnp.pad(W_o[:FA], ((0, 0), (0, HP - H))).astype(f32)
    wo2 = jnp.pad(W_o[FA:], ((0, HP - H), (0, HP - H))).astype(f32)
    bop = jnp.pad(b_o, (0, HP - H)).astype(f32).reshape(1, HP)

    fa_pad = jnp.pad(f_atoms, ((0, A2 - A), (0, 0)))
    a2b_flat = jnp.pad(a2b, ((0, A2 - A), (0, 0))).reshape(A2 * NB)

    u = _t_atom_mm(fa_pad, wi1)
    g = _sc_gather_rows(u, b2a)
    msg = _t_init(g, f_bonds, wi2)

    for _ in range(DEPTH - 1):
        nei_a = _sc_gather_sum(msg, a2b_flat)
        nei_m = _sc_gather_sub(nei_a, msg, b2a, b2revb)
        msg = _t_update(nei_m, whp)

    a_msg = _sc_gather_sum(msg, a2b_flat)
    return _t_final(f_atoms, a_msg, wo1, wo2, bop)
